# Initial kernel scaffold; baseline (speedup 1.0000x reference)
import jax
import jax.numpy as jnp
from jax.experimental import pallas as pl


def kernel(loc_data, conf_data, prior_data):
    raise NotImplementedError("write your pallas kernel here")


# speedup vs baseline: 30.4626x; 30.4626x over previous
import jax
import jax.numpy as jnp
from jax import lax
from jax.experimental import pallas as pl

_N = 20000
_NPAD = 20480
_ROWS = 160
_COLS = 128
_K = 5000
_TOPK = 750
_IOU_T = 0.3
_CONF_T = 0.05
_VAR0 = 0.1
_VAR1 = 0.2
_OUT_ROWS = 768


def _body(sc_ref, lx_ref, ly_ref, lw_ref, lh_ref,
          px_ref, py_ref, pw_ref, ph_ref, out_ref):
    f32 = jnp.float32
    i32 = jnp.int32
    score = sc_ref[...]
    pcx = px_ref[...]
    pcy = py_ref[...]
    pw = pw_ref[...]
    ph = ph_ref[...]

    cx = pcx + lx_ref[...] * f32(_VAR0) * pw
    cy = pcy + ly_ref[...] * f32(_VAR0) * ph
    w = pw * jnp.exp(lw_ref[...] * f32(_VAR1))
    h = ph * jnp.exp(lh_ref[...] * f32(_VAR1))
    x1 = cx - w / f32(2.0)
    y1 = cy - h / f32(2.0)
    x2 = x1 + w
    y2 = y1 + h
    area = (x2 - x1) * (y2 - y1)

    gidx = (lax.broadcasted_iota(i32, (_ROWS, _COLS), 0) * _COLS
            + lax.broadcasted_iota(i32, (_ROWS, _COLS), 1))

    valid = score > f32(_CONF_T)
    key = jnp.where(valid, lax.bitcast_convert_type(score, i32), i32(-1))

    def _bs_val(_, lohi):
        lo, hi = lohi
        mid = lo + (hi - lo) // 2
        c = jnp.sum((key >= mid).astype(i32))
        take = c >= _K
        return (jnp.where(take, mid, lo), jnp.where(take, hi, mid))

    lo, _ = lax.fori_loop(0, 31, _bs_val,
                          (i32(-1), i32(0x7F800000)))
    sstar = lo
    cgt = jnp.sum((key > sstar).astype(i32))
    need = i32(_K) - cgt
    tie = key == sstar

    def _bs_idx(_, lohi):
        lo, hi = lohi
        mid = lo + (hi - lo) // 2
        c = jnp.sum((tie & (gidx >= mid)).astype(i32))
        take = c >= need
        return (jnp.where(take, mid, lo), jnp.where(take, hi, mid))

    lo2, _ = lax.fori_loop(0, 15, _bs_idx, (i32(0), i32(_NPAD)))
    in_top = (key > sstar) | (tie & (gidx >= lo2))

    neg = f32(-jnp.inf)
    msc0 = jnp.where(valid & in_top, score, neg)

    lane = lax.broadcasted_iota(i32, (1, _COLS), 1)

    def _nms(t, msc):
        mx = jnp.max(msc)
        anyact = mx > neg
        pos = jnp.max(jnp.where(msc == mx, gidx, i32(-1)))
        onehot = gidx == pos
        zero = f32(0.0)
        x1p = jnp.sum(jnp.where(onehot, x1, zero))
        y1p = jnp.sum(jnp.where(onehot, y1, zero))
        x2p = jnp.sum(jnp.where(onehot, x2, zero))
        y2p = jnp.sum(jnp.where(onehot, y2, zero))
        areap = (x2p - x1p) * (y2p - y1p)

        xx1 = jnp.maximum(x1, x1p)
        yy1 = jnp.maximum(y1, y1p)
        xx2 = jnp.minimum(x2, x2p)
        yy2 = jnp.minimum(y2, y2p)
        iw = jnp.maximum(xx2 - xx1, zero)
        ih = jnp.maximum(yy2 - yy1, zero)
        inter = iw * ih
        union = area - inter + areap
        iou = inter / union
        keepm = (iou <= f32(_IOU_T)) & (pos != gidx)
        msc = jnp.where(keepm, msc, neg)

        row = jnp.where(lane == 0, mx,
              jnp.where(lane == 1, x1p,
              jnp.where(lane == 2, y1p,
              jnp.where(lane == 3, x2p,
              jnp.where(lane == 4, y2p, zero)))))
        row = jnp.where(anyact, row, zero)
        out_ref[pl.ds(t, 1), :] = row
        return msc

    lax.fori_loop(0, _TOPK, _nms, msc0)


def kernel(loc_data, conf_data, prior_data):
    num = loc_data.shape[0]
    f32 = jnp.float32

    def plane(a):
        return jnp.pad(a.astype(f32), (0, _NPAD - _N)).reshape(_ROWS, _COLS)

    scores = conf_data[0, :, 1]
    loc = loc_data[0]
    args = [plane(scores),
            plane(loc[:, 0]), plane(loc[:, 1]),
            plane(loc[:, 2]), plane(loc[:, 3]),
            plane(prior_data[:, 0]), plane(prior_data[:, 1]),
            plane(prior_data[:, 2]), plane(prior_data[:, 3])]

    res = pl.pallas_call(
        _body,
        out_shape=jax.ShapeDtypeStruct((_OUT_ROWS, _COLS), f32),
    )(*args)

    out = jnp.zeros((num, 2, _TOPK, 5), dtype=f32)
    return out.at[0, 1].set(res[:_TOPK, :5])


# speedup vs baseline: 37.4622x; 1.2298x over previous
import jax
import jax.numpy as jnp
from jax import lax
from jax.experimental import pallas as pl
from jax.experimental.pallas import tpu as pltpu

_N = 20000
_NPAD = 20480
_ROWS = 160
_COLS = 128
_K = 5000
_TOPK = 750
_IOU_T = 0.3
_CONF_T = 0.05
_VAR0 = 0.1
_VAR1 = 0.2
_OUT_ROWS = 768


def _body(sc_ref, lx_ref, ly_ref, lw_ref, lh_ref,
          px_ref, py_ref, pw_ref, ph_ref, out_ref,
          x1_ref, y1_ref, x2_ref, y2_ref, ar_ref):
    f32 = jnp.float32
    i32 = jnp.int32
    score = sc_ref[...]
    pw = pw_ref[...]
    ph = ph_ref[...]

    cx = px_ref[...] + lx_ref[...] * f32(_VAR0) * pw
    cy = py_ref[...] + ly_ref[...] * f32(_VAR0) * ph
    w = pw * jnp.exp(lw_ref[...] * f32(_VAR1))
    h = ph * jnp.exp(lh_ref[...] * f32(_VAR1))
    x1 = cx - w / f32(2.0)
    y1 = cy - h / f32(2.0)
    x2 = x1 + w
    y2 = y1 + h
    x1_ref[...] = x1
    y1_ref[...] = y1
    x2_ref[...] = x2
    y2_ref[...] = y2
    ar_ref[...] = (x2 - x1) * (y2 - y1)
    out_ref[...] = jnp.zeros((_OUT_ROWS, _COLS), f32)

    gidx = (lax.broadcasted_iota(i32, (_ROWS, _COLS), 0) * _COLS
            + lax.broadcasted_iota(i32, (_ROWS, _COLS), 1))

    valid = score > f32(_CONF_T)
    key = jnp.where(valid, lax.bitcast_convert_type(score, i32), i32(-1))

    def _bs_val(_, lohi):
        lo, hi = lohi
        mid = lo + (hi - lo) // 2
        c = jnp.sum((key >= mid).astype(i32))
        take = c >= _K
        return (jnp.where(take, mid, lo), jnp.where(take, hi, mid))

    lo, _ = lax.fori_loop(0, 31, _bs_val,
                          (i32(-1), i32(0x7F800000)))
    sstar = lo
    cgt = jnp.sum((key > sstar).astype(i32))
    need = i32(_K) - cgt
    tie = key == sstar

    def _bs_idx(_, lohi):
        lo, hi = lohi
        mid = lo + (hi - lo) // 2
        c = jnp.sum((tie & (gidx >= mid)).astype(i32))
        take = c >= need
        return (jnp.where(take, mid, lo), jnp.where(take, hi, mid))

    lo2, _ = lax.fori_loop(0, 15, _bs_idx, (i32(0), i32(_NPAD)))
    in_top = (key > sstar) | (tie & (gidx >= lo2))

    neg = f32(-jnp.inf)
    msc0 = jnp.where(valid & in_top, score, neg)

    lane = lax.broadcasted_iota(i32, (1, _COLS), 1)
    zero = f32(0.0)

    def _cond(state):
        t, _, mx = state
        return (t < _TOPK) & (mx > neg)

    def _pick(state):
        t, msc, mx = state
        pos = jnp.max(jnp.where(msc == mx, gidx, i32(-1)))
        r = pos // _COLS
        c = pos - r * _COLS
        loh = lane == c
        x1v = x1_ref[pl.ds(r, 1), :]
        y1v = y1_ref[pl.ds(r, 1), :]
        x2v = x2_ref[pl.ds(r, 1), :]
        y2v = y2_ref[pl.ds(r, 1), :]
        x1p = jnp.sum(jnp.where(loh, x1v, zero))
        y1p = jnp.sum(jnp.where(loh, y1v, zero))
        x2p = jnp.sum(jnp.where(loh, x2v, zero))
        y2p = jnp.sum(jnp.where(loh, y2v, zero))
        areap = (x2p - x1p) * (y2p - y1p)

        iw = jnp.maximum(jnp.minimum(x2_ref[...], x2p)
                         - jnp.maximum(x1_ref[...], x1p), zero)
        ih = jnp.maximum(jnp.minimum(y2_ref[...], y2p)
                         - jnp.maximum(y1_ref[...], y1p), zero)
        inter = iw * ih
        union = ar_ref[...] - inter + areap
        iou = inter / union
        msc = jnp.where(iou <= f32(_IOU_T), msc, neg)

        row = jnp.where(lane == 0, mx,
              jnp.where(lane == 1, x1p,
              jnp.where(lane == 2, y1p,
              jnp.where(lane == 3, x2p,
              jnp.where(lane == 4, y2p, zero)))))
        out_ref[pl.ds(t, 1), :] = row
        return t + 1, msc, jnp.max(msc)

    lax.while_loop(_cond, _pick, (i32(0), msc0, jnp.max(msc0)))


def kernel(loc_data, conf_data, prior_data):
    num = loc_data.shape[0]
    f32 = jnp.float32

    def plane(a):
        return jnp.pad(a.astype(f32), (0, _NPAD - _N)).reshape(_ROWS, _COLS)

    scores = conf_data[0, :, 1]
    loc = loc_data[0]
    args = [plane(scores),
            plane(loc[:, 0]), plane(loc[:, 1]),
            plane(loc[:, 2]), plane(loc[:, 3]),
            plane(prior_data[:, 0]), plane(prior_data[:, 1]),
            plane(prior_data[:, 2]), plane(prior_data[:, 3])]

    res = pl.pallas_call(
        _body,
        out_shape=jax.ShapeDtypeStruct((_OUT_ROWS, _COLS), f32),
        scratch_shapes=[pltpu.VMEM((_ROWS, _COLS), f32)] * 5,
    )(*args)

    out = jnp.zeros((num, 2, _TOPK, 5), dtype=f32)
    return out.at[0, 1].set(res[:_TOPK, :5])
